# baseline (speedup 1.0000x reference)
import jax
import jax.numpy as jnp
from jax import lax
from jax.experimental import pallas as pl
from jax.experimental.pallas import tpu as pltpu
from jax.experimental.pallas import tpu_sc as plsc

_B, _L, _D = 4, 8192, 1024
_ROWS_PER_W = _L // 8
_CR = 8
_NCHUNK = _ROWS_PER_W // _CR
_NBUF = 4
_VREGS_ROW = _D // 16


def _sc_body(x_hbm, pos_hbm, out_hbm, *scratch):
    xb = scratch[0:_NBUF]
    pb = scratch[_NBUF:2 * _NBUF]
    inx = scratch[2 * _NBUF:3 * _NBUF]
    inp = scratch[3 * _NBUF:4 * _NBUF]
    outs = scratch[4 * _NBUF:5 * _NBUF]

    c = lax.axis_index("c")
    s = lax.axis_index("s")
    wid = s * 2 + c
    bi = wid // 8
    row0 = (wid % 8) * _ROWS_PER_W

    def in_copy(i, b):
        r = row0 + i * _CR
        pltpu.async_copy(x_hbm.at[bi, pl.ds(r, _CR), :], xb[b], inx[b])
        pltpu.async_copy(pos_hbm.at[pl.ds(r, _CR), :], pb[b], inp[b])

    def in_wait(i, b):
        r = row0 + i * _CR
        pltpu.make_async_copy(x_hbm.at[bi, pl.ds(r, _CR), :], xb[b], inx[b]).wait()
        pltpu.make_async_copy(pos_hbm.at[pl.ds(r, _CR), :], pb[b], inp[b]).wait()

    def out_copy(i, b):
        r = row0 + i * _CR
        pltpu.async_copy(xb[b], out_hbm.at[bi, pl.ds(r, _CR), :], outs[b])

    def out_wait(i, b):
        r = row0 + i * _CR
        pltpu.make_async_copy(xb[b], out_hbm.at[bi, pl.ds(r, _CR), :], outs[b]).wait()

    def compute(b):
        xref, pref = xb[b], pb[b]

        @plsc.parallel_loop(0, _CR * _VREGS_ROW, 1, unroll=16)
        def add_body(k):
            r = k >> 6
            col = (k & (_VREGS_ROW - 1)) * 16
            plsc.addupdate(xref.at[r, pl.ds(col, 16)], pref[r, pl.ds(col, 16)])

    in_copy(0, 0)
    in_copy(1, 1)

    for b in range(_NBUF):
        i = b
        jp = (b + 2) % _NBUF
        if b < 2:
            in_copy(i + 2, jp)
        else:
            out_wait(i - 2, jp)
            in_copy(i + 2, jp)
        in_wait(i, b)
        compute(b)
        out_copy(i, b)

    def steady(g, carry):
        i0 = g * _NBUF
        for b in range(_NBUF):
            i = i0 + b
            jp = (b + 2) % _NBUF
            out_wait(i - 2, jp)
            in_copy(i + 2, jp)
            in_wait(i, b)
            compute(b)
            out_copy(i, b)
        return carry

    lax.fori_loop(1, _NCHUNK // _NBUF - 1, steady, 0)

    i0 = _NCHUNK - _NBUF
    for b in range(_NBUF):
        i = i0 + b
        jp = (b + 2) % _NBUF
        if b < 2:
            out_wait(i - 2, jp)
            in_copy(i + 2, jp)
        in_wait(i, b)
        compute(b)
        out_copy(i, b)

    for b in range(_NBUF):
        out_wait(i0 + b, b)


def kernel(x, pos_table):
    mesh = plsc.VectorSubcoreMesh(core_axis_name="c", subcore_axis_name="s")
    scratch = (
        [pltpu.VMEM((_CR, _D), jnp.float32)] * (2 * _NBUF)
        + [pltpu.SemaphoreType.DMA] * (3 * _NBUF)
    )
    run = pl.kernel(
        _sc_body,
        mesh=mesh,
        out_type=jax.ShapeDtypeStruct((_B, _L, _D), jnp.float32),
        scratch_types=scratch,
    )
    return run(x, pos_table)


# speedup vs baseline: 1.0015x; 1.0015x over previous
import jax
import jax.numpy as jnp
from jax import lax
from jax.experimental import pallas as pl
from jax.experimental.pallas import tpu as pltpu
from jax.experimental.pallas import tpu_sc as plsc

_B, _L, _D = 4, 8192, 1024
_ROWS_PER_W = _L // 8
_CR = 16
_NCHUNK = _ROWS_PER_W // _CR
_NGROUP = _NCHUNK // 2
_VREGS = _CR * (_D // 16)


def _sc_body(x_hbm, pos_hbm, out_hbm, *scratch):
    xb = scratch[0:2]
    pb = scratch[2:4]
    ob = scratch[4:6]
    inx = scratch[6:8]
    inp = scratch[8:10]
    outs = scratch[10:12]

    c = lax.axis_index("c")
    s = lax.axis_index("s")
    wid = s * 2 + c
    bi = wid // 8
    row0 = (wid % 8) * _ROWS_PER_W

    def in_copy(i, b):
        r = row0 + i * _CR
        pltpu.async_copy(x_hbm.at[bi, pl.ds(r, _CR), :], xb[b], inx[b])
        pltpu.async_copy(pos_hbm.at[pl.ds(r, _CR), :], pb[b], inp[b])

    def in_wait(i, b):
        r = row0 + i * _CR
        pltpu.make_async_copy(x_hbm.at[bi, pl.ds(r, _CR), :], xb[b], inx[b]).wait()
        pltpu.make_async_copy(pos_hbm.at[pl.ds(r, _CR), :], pb[b], inp[b]).wait()

    def out_copy(i, b):
        r = row0 + i * _CR
        pltpu.async_copy(ob[b], out_hbm.at[bi, pl.ds(r, _CR), :], outs[b])

    def out_wait(i, b):
        r = row0 + i * _CR
        pltpu.make_async_copy(ob[b], out_hbm.at[bi, pl.ds(r, _CR), :], outs[b]).wait()

    def compute(b):
        xref, pref, oref = xb[b], pb[b], ob[b]

        @plsc.parallel_loop(0, _VREGS, 1, unroll=16)
        def add_body(k):
            r = k >> 6
            col = (k & 63) * 16
            oref[r, pl.ds(col, 16)] = (
                xref[r, pl.ds(col, 16)] + pref[r, pl.ds(col, 16)]
            )

    in_copy(0, 0)
    in_copy(1, 1)

    def group(g, carry):
        for b in range(2):
            i = 2 * g + b
            in_wait(i, b)

            @pl.when(g >= 1)
            def _():
                out_wait(i - 2, b)

            compute(b)
            out_copy(i, b)

            @pl.when(g <= _NGROUP - 2)
            def _():
                in_copy(i + 2, b)

        return carry

    lax.fori_loop(0, _NGROUP, group, 0)

    out_wait(_NCHUNK - 2, 0)
    out_wait(_NCHUNK - 1, 1)


def kernel(x, pos_table):
    mesh = plsc.VectorSubcoreMesh(core_axis_name="c", subcore_axis_name="s")
    scratch = (
        [pltpu.VMEM((_CR, _D), jnp.float32)] * 6
        + [pltpu.SemaphoreType.DMA] * 6
    )
    run = pl.kernel(
        _sc_body,
        mesh=mesh,
        out_type=jax.ShapeDtypeStruct((_B, _L, _D), jnp.float32),
        scratch_types=scratch,
    )
    return run(x, pos_table)


# speedup vs baseline: 1.7283x; 1.7257x over previous
import jax
import jax.numpy as jnp
from jax.experimental import pallas as pl


def _add_body(x_ref, pos_ref, out_ref):
    out_ref[...] = x_ref[...] + pos_ref[...]


def kernel(x, pos_table):
    B, L, D = x.shape
    BL = 2048
    num_l = L // BL
    grid = (num_l, B)
    return pl.pallas_call(
        _add_body,
        grid=grid,
        in_specs=[
            pl.BlockSpec((1, BL, D), lambda l, b: (b, l, 0)),
            pl.BlockSpec((BL, D), lambda l, b: (l, 0)),
        ],
        out_specs=pl.BlockSpec((1, BL, D), lambda l, b: (b, l, 0)),
        out_shape=jax.ShapeDtypeStruct((B, L, D), x.dtype),
    )(x, pos_table)
